# baseline (speedup 1.0000x reference)
import functools

import jax
import jax.numpy as jnp
from jax import lax
from jax.experimental import pallas as pl
from jax.experimental.pallas import tpu as pltpu
from jax.experimental.pallas import tpu_sc as plsc

N = 50000
D = 64
EDGES = 800000

CH = 128
NSUB = 16
NCORE = 2
EPAD = 823296
NCHUNKS = EPAD // CH
CPT = NCHUNKS // NSUB
HALF = 25000
ACC_R = 25088
STRIPE = ACC_R // NSUB
DUMMY = HALF
DEG_R = 51200
DEG_CPT = NCHUNKS // (NSUB * NCORE)

_mesh = plsc.VectorSubcoreMesh(core_axis_name="c", subcore_axis_name="s")


def _fill1d(ref, n, value):
    vec = jnp.full((16,), value, jnp.float32)

    def body(i, _):
        ref[pl.ds(i * 16, 16)] = vec
        return 0

    lax.fori_loop(0, n // 16, body, 0)


@functools.partial(
    pl.kernel,
    out_type=jax.ShapeDtypeStruct((NCORE * DEG_R,), jnp.float32),
    mesh=_mesh,
    scratch_types=[
        pltpu.VMEM((CH,), jnp.int32),
        pltpu.VMEM((CH,), jnp.float32),
        pltpu.VMEM((1600,), jnp.float32),
        pltpu.VMEM_SHARED((DEG_R,), jnp.float32),
    ],
)
def _sc_deg(col_hbm, out_hbm, col_v, ones_v, stage_v, deg_sh):
    c = lax.axis_index("c")
    s = lax.axis_index("s")
    _fill1d(ones_v, CH, 1.0)
    _fill1d(stage_v, 1600, 0.0)
    for t in range(2):
        pltpu.sync_copy(stage_v, deg_sh.at[pl.ds(s * 3200 + t * 1600, 1600)])
    plsc.subcore_barrier()
    base_edge = (c * (NSUB * DEG_CPT) + s * DEG_CPT) * CH

    def body(j, _):
        pltpu.sync_copy(col_hbm.at[pl.ds(base_edge + j * CH, CH)], col_v)
        pltpu.sync_copy(ones_v, deg_sh.at[col_v], add=True)
        return 0

    lax.fori_loop(0, DEG_CPT, body, 0)
    plsc.subcore_barrier()
    for t in range(2):
        off = s * 3200 + t * 1600
        pltpu.sync_copy(deg_sh.at[pl.ds(off, 1600)], stage_v)
        pltpu.sync_copy(stage_v, out_hbm.at[pl.ds(c * DEG_R + off, 1600)])


@functools.partial(
    pl.kernel,
    out_type=jax.ShapeDtypeStruct((NCORE, ACC_R, D), jnp.float32),
    mesh=_mesh,
    scratch_types=[
        pltpu.VMEM((CH,), jnp.int32),
        pltpu.VMEM((CH,), jnp.int32),
        pltpu.VMEM((CH, D), jnp.float32),
        pltpu.VMEM_SHARED((ACC_R, D), jnp.float32),
        pltpu.SemaphoreType.DMA,
    ],
    compiler_params=pltpu.CompilerParams(use_tc_tiling_on_sc=False),
)
def _sc_agg(row_hbm, col_hbm, z_hbm, out_hbm, idxr, idxc, rows_v, acc_sh, gsem):
    c = lax.axis_index("c")
    s = lax.axis_index("s")

    def zrow(i, _):
        for k in range(D // 16):
            rows_v[i, pl.ds(k * 16, 16)] = jnp.zeros((16,), jnp.float32)
        return 0

    lax.fori_loop(0, CH, zrow, 0)
    for q in range(STRIPE // CH):
        pltpu.sync_copy(rows_v, acc_sh.at[pl.ds(s * STRIPE + q * CH, CH)])
    pltpu.sync_copy(rows_v.at[pl.ds(0, STRIPE % CH)],
                    acc_sh.at[pl.ds(s * STRIPE + (STRIPE // CH) * CH,
                                    STRIPE % CH)])
    plsc.subcore_barrier()

    lo = c * HALF

    def jbody(j, _):
        b = (s * CPT + j) * CH
        pltpu.sync_copy(row_hbm.at[pl.ds(b, CH)], idxr)
        pltpu.sync_copy(col_hbm.at[pl.ds(b, CH)], idxc)
        for k in range(CH // 16):
            cv = idxc[pl.ds(k * 16, 16)]
            lc = cv - lo
            ok = (lc >= 0) & (lc < HALF)
            idxc[pl.ds(k * 16, 16)] = jnp.where(ok, lc, DUMMY)
        pltpu.async_copy(z_hbm.at[idxr], rows_v, gsem).wait()
        pltpu.sync_copy(rows_v, acc_sh.at[idxc], add=True)
        return 0

    lax.fori_loop(0, CPT, jbody, 0)

    plsc.subcore_barrier()
    for q in range(STRIPE // CH):
        st = s * STRIPE + q * CH
        pltpu.sync_copy(acc_sh.at[pl.ds(st, CH)], rows_v)
        pltpu.sync_copy(rows_v, out_hbm.at[c, pl.ds(st, CH)])
    st = s * STRIPE + (STRIPE // CH) * CH
    pltpu.sync_copy(acc_sh.at[pl.ds(st, STRIPE % CH)],
                    rows_v.at[pl.ds(0, STRIPE % CH)])
    pltpu.sync_copy(rows_v.at[pl.ds(0, STRIPE % CH)],
                    out_hbm.at[c, pl.ds(st, STRIPE % CH)])


BLK = 1000
GRID = N // BLK


def _tc_pre_body(e_ref, d0_ref, d1_ref, wt_ref, b_ref, dinv_ref, z_ref):
    dinv = lax.rsqrt(d0_ref[...] + d1_ref[...] + 1.0)
    y = jnp.dot(e_ref[...], wt_ref[...], preferred_element_type=jnp.float32)
    dinv_ref[...] = dinv
    z_ref[...] = dinv * (y + b_ref[...])


def _tc_mid_body(acc_ref, z_ref, dinv_ref, wt_ref, b_ref, e_ref, zn_ref):
    o = dinv_ref[...] * (acc_ref[...] + z_ref[...])
    o = jnp.where(o >= 0, o, 0.01 * o)
    nrm = jnp.sqrt(jnp.sum(o * o, axis=1, keepdims=True))
    o = o / jnp.maximum(nrm, 1e-12)
    e_ref[...] = o
    zn_ref[...] = dinv_ref[...] * (
        jnp.dot(o, wt_ref[...], preferred_element_type=jnp.float32) + b_ref[...])


def _tc_post_body(acc_ref, z_ref, dinv_ref, e_ref):
    o = dinv_ref[...] * (acc_ref[...] + z_ref[...])
    o = jnp.where(o >= 0, o, 0.01 * o)
    nrm = jnp.sqrt(jnp.sum(o * o, axis=1, keepdims=True))
    e_ref[...] = o / jnp.maximum(nrm, 1e-12)


_row_spec = pl.BlockSpec((BLK, D), lambda i: (i, 0))
_col1_spec = pl.BlockSpec((BLK, 1), lambda i: (i, 0))
_w_spec = pl.BlockSpec((D, D), lambda i: (0, 0))
_b_spec = pl.BlockSpec((1, D), lambda i: (0, 0))

_tc_pre = pl.pallas_call(
    _tc_pre_body,
    grid=(GRID,),
    in_specs=[_row_spec, _col1_spec, _col1_spec, _w_spec, _b_spec],
    out_specs=[_col1_spec, _row_spec],
    out_shape=[
        jax.ShapeDtypeStruct((N, 1), jnp.float32),
        jax.ShapeDtypeStruct((N, D), jnp.float32),
    ],
)

_tc_mid = pl.pallas_call(
    _tc_mid_body,
    grid=(GRID,),
    in_specs=[_row_spec, _row_spec, _col1_spec, _w_spec, _b_spec],
    out_specs=[_row_spec, _row_spec],
    out_shape=[
        jax.ShapeDtypeStruct((N, D), jnp.float32),
        jax.ShapeDtypeStruct((N, D), jnp.float32),
    ],
)

_tc_post = pl.pallas_call(
    _tc_post_body,
    grid=(GRID,),
    in_specs=[_row_spec, _row_spec, _col1_spec],
    out_specs=_row_spec,
    out_shape=jax.ShapeDtypeStruct((N, D), jnp.float32),
)


def kernel(edge_index, E, W1, b1, W2, b2, W3, b3):
    ei = edge_index.astype(jnp.int32)
    pad = EPAD - EDGES
    row = jnp.concatenate([ei[0], jnp.zeros((pad,), jnp.int32)])
    col = jnp.concatenate([ei[1], jnp.full((pad,), N, jnp.int32)])

    deg2 = _sc_deg(col)
    d0 = deg2[:N, None]
    d1 = deg2[DEG_R:DEG_R + N, None]

    dinv, z1 = _tc_pre(E, d0, d1, W1.T, b1[None, :])

    def agg(z):
        a = _sc_agg(row, col, z)
        return jnp.concatenate([a[0, :HALF], a[1, :HALF]], axis=0)

    E1, z2 = _tc_mid(agg(z1), z1, dinv, W2.T, b2[None, :])
    E2, z3 = _tc_mid(agg(z2), z2, dinv, W3.T, b3[None, :])
    E3 = _tc_post(agg(z3), z3, dinv)
    return jnp.concatenate([E, E1, E2, E3], axis=1)


# speedup vs baseline: 1.1094x; 1.1094x over previous
import functools

import jax
import jax.numpy as jnp
from jax import lax
from jax.experimental import pallas as pl
from jax.experimental.pallas import tpu as pltpu
from jax.experimental.pallas import tpu_sc as plsc

N = 50000
D = 64
EDGES = 800000

CH = 128
NSUB = 16
NCORE = 2
EPAD = 823296
NCHUNKS = EPAD // CH
CPT = NCHUNKS // NSUB
HALF = 25000
ACC_R = 25088
STRIPE = ACC_R // NSUB
DUMMY = HALF
DEG_R = 51200
DEG_CPT = NCHUNKS // (NSUB * NCORE)

_mesh = plsc.VectorSubcoreMesh(core_axis_name="c", subcore_axis_name="s")


def _fill1d(ref, n, value):
    vec = jnp.full((16,), value, jnp.float32)

    def body(i, _):
        ref[pl.ds(i * 16, 16)] = vec
        return 0

    lax.fori_loop(0, n // 16, body, 0)


@functools.partial(
    pl.kernel,
    out_type=jax.ShapeDtypeStruct((NCORE * DEG_R,), jnp.float32),
    mesh=_mesh,
    scratch_types=[
        pltpu.VMEM((CH,), jnp.int32),
        pltpu.VMEM((CH,), jnp.float32),
        pltpu.VMEM((1600,), jnp.float32),
        pltpu.VMEM_SHARED((DEG_R,), jnp.float32),
    ],
)
def _sc_deg(col_hbm, out_hbm, col_v, ones_v, stage_v, deg_sh):
    c = lax.axis_index("c")
    s = lax.axis_index("s")
    _fill1d(ones_v, CH, 1.0)
    _fill1d(stage_v, 1600, 0.0)
    for t in range(2):
        pltpu.sync_copy(stage_v, deg_sh.at[pl.ds(s * 3200 + t * 1600, 1600)])
    plsc.subcore_barrier()
    base_edge = (c * (NSUB * DEG_CPT) + s * DEG_CPT) * CH

    def body(j, _):
        pltpu.sync_copy(col_hbm.at[pl.ds(base_edge + j * CH, CH)], col_v)
        pltpu.sync_copy(ones_v, deg_sh.at[col_v], add=True)
        return 0

    lax.fori_loop(0, DEG_CPT, body, 0)
    plsc.subcore_barrier()
    for t in range(2):
        off = s * 3200 + t * 1600
        pltpu.sync_copy(deg_sh.at[pl.ds(off, 1600)], stage_v)
        pltpu.sync_copy(stage_v, out_hbm.at[pl.ds(c * DEG_R + off, 1600)])


@functools.partial(
    pl.kernel,
    out_type=jax.ShapeDtypeStruct((NCORE, ACC_R, D), jnp.float32),
    mesh=_mesh,
    scratch_types=[
        pltpu.VMEM((CH,), jnp.int32),
        pltpu.VMEM((CH,), jnp.int32),
        pltpu.VMEM((CH,), jnp.int32),
        pltpu.VMEM((CH,), jnp.int32),
        pltpu.VMEM((CH, D), jnp.float32),
        pltpu.VMEM((CH, D), jnp.float32),
        pltpu.VMEM_SHARED((ACC_R, D), jnp.float32),
        pltpu.SemaphoreType.DMA,
        pltpu.SemaphoreType.DMA,
    ],
    compiler_params=pltpu.CompilerParams(use_tc_tiling_on_sc=False),
)
def _sc_agg(row_hbm, col_hbm, z_hbm, out_hbm,
            idxr0, idxc0, idxr1, idxc1, rows0, rows1, acc_sh, sem0, sem1):
    c = lax.axis_index("c")
    s = lax.axis_index("s")

    def zrow(i, _):
        for k in range(D // 16):
            rows0[i, pl.ds(k * 16, 16)] = jnp.zeros((16,), jnp.float32)
        return 0

    lax.fori_loop(0, CH, zrow, 0)
    for q in range(STRIPE // CH):
        pltpu.sync_copy(rows0, acc_sh.at[pl.ds(s * STRIPE + q * CH, CH)])
    pltpu.sync_copy(rows0.at[pl.ds(0, STRIPE % CH)],
                    acc_sh.at[pl.ds(s * STRIPE + (STRIPE // CH) * CH,
                                    STRIPE % CH)])
    plsc.subcore_barrier()

    lo = c * HALF

    def remap(idxc):
        for k in range(CH // 16):
            cv = idxc[pl.ds(k * 16, 16)]
            lc = cv - lo
            ok = (lc >= 0) & (lc < HALF)
            idxc[pl.ds(k * 16, 16)] = jnp.where(ok, lc, DUMMY)

    def jbody(j, _):
        a = (s * CPT + 2 * j) * CH
        pltpu.sync_copy(row_hbm.at[pl.ds(a, CH)], idxr0)
        pltpu.sync_copy(col_hbm.at[pl.ds(a, CH)], idxc0)
        remap(idxc0)
        h0 = pltpu.async_copy(z_hbm.at[idxr0], rows0, sem0)
        pltpu.sync_copy(row_hbm.at[pl.ds(a + CH, CH)], idxr1)
        pltpu.sync_copy(col_hbm.at[pl.ds(a + CH, CH)], idxc1)
        remap(idxc1)
        h1 = pltpu.async_copy(z_hbm.at[idxr1], rows1, sem1)
        h0.wait()
        pltpu.sync_copy(rows0, acc_sh.at[idxc0], add=True)
        h1.wait()
        pltpu.sync_copy(rows1, acc_sh.at[idxc1], add=True)
        return 0

    lax.fori_loop(0, CPT // 2, jbody, 0)

    plsc.subcore_barrier()
    for q in range(STRIPE // CH):
        st = s * STRIPE + q * CH
        pltpu.sync_copy(acc_sh.at[pl.ds(st, CH)], rows0)
        pltpu.sync_copy(rows0, out_hbm.at[c, pl.ds(st, CH)])
    st = s * STRIPE + (STRIPE // CH) * CH
    pltpu.sync_copy(acc_sh.at[pl.ds(st, STRIPE % CH)],
                    rows0.at[pl.ds(0, STRIPE % CH)])
    pltpu.sync_copy(rows0.at[pl.ds(0, STRIPE % CH)],
                    out_hbm.at[c, pl.ds(st, STRIPE % CH)])


BLK = 1000
GRID = N // BLK


def _tc_pre_body(e_ref, d0_ref, d1_ref, wt_ref, b_ref, dinv_ref, z_ref):
    dinv = lax.rsqrt(d0_ref[...] + d1_ref[...] + 1.0)
    y = jnp.dot(e_ref[...], wt_ref[...], preferred_element_type=jnp.float32)
    dinv_ref[...] = dinv
    z_ref[...] = dinv * (y + b_ref[...])


def _tc_mid_body(acc_ref, z_ref, dinv_ref, wt_ref, b_ref, e_ref, zn_ref):
    o = dinv_ref[...] * (acc_ref[...] + z_ref[...])
    o = jnp.where(o >= 0, o, 0.01 * o)
    nrm = jnp.sqrt(jnp.sum(o * o, axis=1, keepdims=True))
    o = o / jnp.maximum(nrm, 1e-12)
    e_ref[...] = o
    zn_ref[...] = dinv_ref[...] * (
        jnp.dot(o, wt_ref[...], preferred_element_type=jnp.float32) + b_ref[...])


def _tc_post_body(acc_ref, z_ref, dinv_ref, e_ref):
    o = dinv_ref[...] * (acc_ref[...] + z_ref[...])
    o = jnp.where(o >= 0, o, 0.01 * o)
    nrm = jnp.sqrt(jnp.sum(o * o, axis=1, keepdims=True))
    e_ref[...] = o / jnp.maximum(nrm, 1e-12)


_row_spec = pl.BlockSpec((BLK, D), lambda i: (i, 0))
_col1_spec = pl.BlockSpec((BLK, 1), lambda i: (i, 0))
_w_spec = pl.BlockSpec((D, D), lambda i: (0, 0))
_b_spec = pl.BlockSpec((1, D), lambda i: (0, 0))

_tc_pre = pl.pallas_call(
    _tc_pre_body,
    grid=(GRID,),
    in_specs=[_row_spec, _col1_spec, _col1_spec, _w_spec, _b_spec],
    out_specs=[_col1_spec, _row_spec],
    out_shape=[
        jax.ShapeDtypeStruct((N, 1), jnp.float32),
        jax.ShapeDtypeStruct((N, D), jnp.float32),
    ],
)

_tc_mid = pl.pallas_call(
    _tc_mid_body,
    grid=(GRID,),
    in_specs=[_row_spec, _row_spec, _col1_spec, _w_spec, _b_spec],
    out_specs=[_row_spec, _row_spec],
    out_shape=[
        jax.ShapeDtypeStruct((N, D), jnp.float32),
        jax.ShapeDtypeStruct((N, D), jnp.float32),
    ],
)

_tc_post = pl.pallas_call(
    _tc_post_body,
    grid=(GRID,),
    in_specs=[_row_spec, _row_spec, _col1_spec],
    out_specs=_row_spec,
    out_shape=jax.ShapeDtypeStruct((N, D), jnp.float32),
)


def kernel(edge_index, E, W1, b1, W2, b2, W3, b3):
    ei = edge_index.astype(jnp.int32)
    pad = EPAD - EDGES
    row = jnp.concatenate([ei[0], jnp.zeros((pad,), jnp.int32)])
    col = jnp.concatenate([ei[1], jnp.full((pad,), N, jnp.int32)])

    deg2 = _sc_deg(col)
    d0 = deg2[:N, None]
    d1 = deg2[DEG_R:DEG_R + N, None]

    dinv, z1 = _tc_pre(E, d0, d1, W1.T, b1[None, :])

    def agg(z):
        a = _sc_agg(row, col, z)
        return jnp.concatenate([a[0, :HALF], a[1, :HALF]], axis=0)

    E1, z2 = _tc_mid(agg(z1), z1, dinv, W2.T, b2[None, :])
    E2, z3 = _tc_mid(agg(z2), z2, dinv, W3.T, b3[None, :])
    E3 = _tc_post(agg(z3), z3, dinv)
    return jnp.concatenate([E, E1, E2, E3], axis=1)


# speedup vs baseline: 1.1536x; 1.0399x over previous
import functools

import jax
import jax.numpy as jnp
from jax import lax
from jax.experimental import pallas as pl
from jax.experimental.pallas import tpu as pltpu
from jax.experimental.pallas import tpu_sc as plsc

N = 50000
D = 64
EDGES = 800000

CH = 128
NSUB = 16
NCORE = 2
EPAD = 823296
IBLK = 6
NCHUNKS = EPAD // CH
CPT = NCHUNKS // NSUB
HALF = 25000
ACC_R = 25088
STRIPE = ACC_R // NSUB
DUMMY = HALF
DEG_R = 51200
DEG_CPT = NCHUNKS // (NSUB * NCORE)

_mesh = plsc.VectorSubcoreMesh(core_axis_name="c", subcore_axis_name="s")


def _fill1d(ref, n, value):
    vec = jnp.full((16,), value, jnp.float32)

    def body(i, _):
        ref[pl.ds(i * 16, 16)] = vec
        return 0

    lax.fori_loop(0, n // 16, body, 0)


@functools.partial(
    pl.kernel,
    out_type=jax.ShapeDtypeStruct((NCORE * DEG_R,), jnp.float32),
    mesh=_mesh,
    scratch_types=[
        pltpu.VMEM((CH,), jnp.int32),
        pltpu.VMEM((CH,), jnp.float32),
        pltpu.VMEM((1600,), jnp.float32),
        pltpu.VMEM_SHARED((DEG_R,), jnp.float32),
    ],
)
def _sc_deg(col_hbm, out_hbm, col_v, ones_v, stage_v, deg_sh):
    c = lax.axis_index("c")
    s = lax.axis_index("s")
    _fill1d(ones_v, CH, 1.0)
    _fill1d(stage_v, 1600, 0.0)
    for t in range(2):
        pltpu.sync_copy(stage_v, deg_sh.at[pl.ds(s * 3200 + t * 1600, 1600)])
    plsc.subcore_barrier()
    base_edge = (c * (NSUB * DEG_CPT) + s * DEG_CPT) * CH

    def body(j, _):
        pltpu.sync_copy(col_hbm.at[pl.ds(base_edge + j * CH, CH)], col_v)
        pltpu.sync_copy(ones_v, deg_sh.at[col_v], add=True)
        return 0

    lax.fori_loop(0, DEG_CPT, body, 0)
    plsc.subcore_barrier()
    for t in range(2):
        off = s * 3200 + t * 1600
        pltpu.sync_copy(deg_sh.at[pl.ds(off, 1600)], stage_v)
        pltpu.sync_copy(stage_v, out_hbm.at[pl.ds(c * DEG_R + off, 1600)])


@functools.partial(
    pl.kernel,
    out_type=jax.ShapeDtypeStruct((NCORE, ACC_R, D), jnp.float32),
    mesh=_mesh,
    scratch_types=[
        pltpu.VMEM((IBLK * CH,), jnp.int32),
        pltpu.VMEM((IBLK * CH,), jnp.int32),
        pltpu.VMEM((CH, D), jnp.float32),
        pltpu.VMEM((CH, D), jnp.float32),
        pltpu.VMEM_SHARED((ACC_R, D), jnp.float32),
        pltpu.SemaphoreType.DMA,
        pltpu.SemaphoreType.DMA,
    ],
    compiler_params=pltpu.CompilerParams(use_tc_tiling_on_sc=False),
)
def _sc_agg(row_hbm, col_hbm, z_hbm, out_hbm,
            idxrb, idxcb, rows0, rows1, acc_sh, sem0, sem1):
    c = lax.axis_index("c")
    s = lax.axis_index("s")

    def zrow(i, _):
        for k in range(D // 16):
            rows0[i, pl.ds(k * 16, 16)] = jnp.zeros((16,), jnp.float32)
        return 0

    lax.fori_loop(0, CH, zrow, 0)
    for q in range(STRIPE // CH):
        pltpu.sync_copy(rows0, acc_sh.at[pl.ds(s * STRIPE + q * CH, CH)])
    pltpu.sync_copy(rows0.at[pl.ds(0, STRIPE % CH)],
                    acc_sh.at[pl.ds(s * STRIPE + (STRIPE // CH) * CH,
                                    STRIPE % CH)])
    plsc.subcore_barrier()

    lo = c * HALF

    def jbody(j, _):
        b = (s * CPT + j * IBLK) * CH
        pltpu.sync_copy(row_hbm.at[pl.ds(b, IBLK * CH)], idxrb)
        pltpu.sync_copy(col_hbm.at[pl.ds(b, IBLK * CH)], idxcb)

        def rbody(k, _):
            cv = idxcb[pl.ds(k * 16, 16)]
            lc = cv - lo
            ok = (lc >= 0) & (lc < HALF)
            idxcb[pl.ds(k * 16, 16)] = jnp.where(ok, lc, DUMMY)
            return 0

        lax.fori_loop(0, IBLK * CH // 16, rbody, 0)
        for p in range(IBLK // 2):
            a0 = (2 * p) * CH
            a1 = (2 * p + 1) * CH
            h0 = pltpu.async_copy(z_hbm.at[idxrb.at[pl.ds(a0, CH)]],
                                  rows0, sem0)
            h1 = pltpu.async_copy(z_hbm.at[idxrb.at[pl.ds(a1, CH)]],
                                  rows1, sem1)
            h0.wait()
            pltpu.sync_copy(rows0, acc_sh.at[idxcb.at[pl.ds(a0, CH)]],
                            add=True)
            h1.wait()
            pltpu.sync_copy(rows1, acc_sh.at[idxcb.at[pl.ds(a1, CH)]],
                            add=True)
        return 0

    lax.fori_loop(0, CPT // IBLK, jbody, 0)

    plsc.subcore_barrier()
    for q in range(STRIPE // CH):
        st = s * STRIPE + q * CH
        pltpu.sync_copy(acc_sh.at[pl.ds(st, CH)], rows0)
        pltpu.sync_copy(rows0, out_hbm.at[c, pl.ds(st, CH)])
    st = s * STRIPE + (STRIPE // CH) * CH
    pltpu.sync_copy(acc_sh.at[pl.ds(st, STRIPE % CH)],
                    rows0.at[pl.ds(0, STRIPE % CH)])
    pltpu.sync_copy(rows0.at[pl.ds(0, STRIPE % CH)],
                    out_hbm.at[c, pl.ds(st, STRIPE % CH)])


BLK = 1000
GRID = N // BLK


def _tc_pre_body(e_ref, d0_ref, d1_ref, wt_ref, b_ref, dinv_ref, z_ref):
    dinv = lax.rsqrt(d0_ref[...] + d1_ref[...] + 1.0)
    y = jnp.dot(e_ref[...], wt_ref[...], preferred_element_type=jnp.float32)
    dinv_ref[...] = dinv
    z_ref[...] = dinv * (y + b_ref[...])


def _tc_mid_body(acc_ref, z_ref, dinv_ref, wt_ref, b_ref, e_ref, zn_ref):
    o = dinv_ref[...] * (acc_ref[...] + z_ref[...])
    o = jnp.where(o >= 0, o, 0.01 * o)
    nrm = jnp.sqrt(jnp.sum(o * o, axis=1, keepdims=True))
    o = o / jnp.maximum(nrm, 1e-12)
    e_ref[...] = o
    zn_ref[...] = dinv_ref[...] * (
        jnp.dot(o, wt_ref[...], preferred_element_type=jnp.float32) + b_ref[...])


def _tc_post_body(acc_ref, z_ref, dinv_ref, e_ref):
    o = dinv_ref[...] * (acc_ref[...] + z_ref[...])
    o = jnp.where(o >= 0, o, 0.01 * o)
    nrm = jnp.sqrt(jnp.sum(o * o, axis=1, keepdims=True))
    e_ref[...] = o / jnp.maximum(nrm, 1e-12)


_row_spec = pl.BlockSpec((BLK, D), lambda i: (i, 0))
_col1_spec = pl.BlockSpec((BLK, 1), lambda i: (i, 0))
_w_spec = pl.BlockSpec((D, D), lambda i: (0, 0))
_b_spec = pl.BlockSpec((1, D), lambda i: (0, 0))

_tc_pre = pl.pallas_call(
    _tc_pre_body,
    grid=(GRID,),
    in_specs=[_row_spec, _col1_spec, _col1_spec, _w_spec, _b_spec],
    out_specs=[_col1_spec, _row_spec],
    out_shape=[
        jax.ShapeDtypeStruct((N, 1), jnp.float32),
        jax.ShapeDtypeStruct((N, D), jnp.float32),
    ],
)

_tc_mid = pl.pallas_call(
    _tc_mid_body,
    grid=(GRID,),
    in_specs=[_row_spec, _row_spec, _col1_spec, _w_spec, _b_spec],
    out_specs=[_row_spec, _row_spec],
    out_shape=[
        jax.ShapeDtypeStruct((N, D), jnp.float32),
        jax.ShapeDtypeStruct((N, D), jnp.float32),
    ],
)

_tc_post = pl.pallas_call(
    _tc_post_body,
    grid=(GRID,),
    in_specs=[_row_spec, _row_spec, _col1_spec],
    out_specs=_row_spec,
    out_shape=jax.ShapeDtypeStruct((N, D), jnp.float32),
)


def kernel(edge_index, E, W1, b1, W2, b2, W3, b3):
    ei = edge_index.astype(jnp.int32)
    pad = EPAD - EDGES
    row = jnp.concatenate([ei[0], jnp.zeros((pad,), jnp.int32)])
    col = jnp.concatenate([ei[1], jnp.full((pad,), N, jnp.int32)])

    deg2 = _sc_deg(col)
    d0 = deg2[:N, None]
    d1 = deg2[DEG_R:DEG_R + N, None]

    dinv, z1 = _tc_pre(E, d0, d1, W1.T, b1[None, :])

    def agg(z):
        a = _sc_agg(row, col, z)
        return jnp.concatenate([a[0, :HALF], a[1, :HALF]], axis=0)

    E1, z2 = _tc_mid(agg(z1), z1, dinv, W2.T, b2[None, :])
    E2, z3 = _tc_mid(agg(z2), z2, dinv, W3.T, b3[None, :])
    E3 = _tc_post(agg(z3), z3, dinv)
    return jnp.concatenate([E, E1, E2, E3], axis=1)


# speedup vs baseline: 1.1649x; 1.0097x over previous
import functools

import jax
import jax.numpy as jnp
from jax import lax
from jax.experimental import pallas as pl
from jax.experimental.pallas import tpu as pltpu
from jax.experimental.pallas import tpu_sc as plsc

N = 50000
D = 64
EDGES = 800000

CH = 128
NSUB = 16
NCORE = 2
EPAD = 823296
IBLK = 6
NCHUNKS = EPAD // CH
CPT = NCHUNKS // NSUB
HALF = 25000
ACC_R = 25088
STRIPE = ACC_R // NSUB
DUMMY = HALF
DEG_R = 51200
DEG_CPT = NCHUNKS // (NSUB * NCORE)

_mesh = plsc.VectorSubcoreMesh(core_axis_name="c", subcore_axis_name="s")


def _fill1d(ref, n, value):
    vec = jnp.full((16,), value, jnp.float32)

    def body(i, _):
        ref[pl.ds(i * 16, 16)] = vec
        return 0

    lax.fori_loop(0, n // 16, body, 0)


@functools.partial(
    pl.kernel,
    out_type=jax.ShapeDtypeStruct((NCORE * DEG_R,), jnp.float32),
    mesh=_mesh,
    scratch_types=[
        pltpu.VMEM((CH,), jnp.int32),
        pltpu.VMEM((CH,), jnp.float32),
        pltpu.VMEM((1600,), jnp.float32),
        pltpu.VMEM_SHARED((DEG_R,), jnp.float32),
    ],
)
def _sc_deg(col_hbm, out_hbm, col_v, ones_v, stage_v, deg_sh):
    c = lax.axis_index("c")
    s = lax.axis_index("s")
    _fill1d(ones_v, CH, 1.0)
    _fill1d(stage_v, 1600, 0.0)
    for t in range(2):
        pltpu.sync_copy(stage_v, deg_sh.at[pl.ds(s * 3200 + t * 1600, 1600)])
    plsc.subcore_barrier()
    base_edge = (c * (NSUB * DEG_CPT) + s * DEG_CPT) * CH

    def body(j, _):
        pltpu.sync_copy(col_hbm.at[pl.ds(base_edge + j * CH, CH)], col_v)
        pltpu.sync_copy(ones_v, deg_sh.at[col_v], add=True)
        return 0

    lax.fori_loop(0, DEG_CPT, body, 0)
    plsc.subcore_barrier()
    for t in range(2):
        off = s * 3200 + t * 1600
        pltpu.sync_copy(deg_sh.at[pl.ds(off, 1600)], stage_v)
        pltpu.sync_copy(stage_v, out_hbm.at[pl.ds(c * DEG_R + off, 1600)])


@functools.partial(
    pl.kernel,
    out_type=jax.ShapeDtypeStruct((NCORE, ACC_R, D), jnp.float32),
    mesh=_mesh,
    scratch_types=[
        pltpu.VMEM((IBLK * CH,), jnp.int32),
        pltpu.VMEM((IBLK * CH,), jnp.int32),
        pltpu.VMEM((CH, D), jnp.float32),
        pltpu.VMEM((CH, D), jnp.float32),
        pltpu.VMEM_SHARED((ACC_R, D), jnp.float32),
        pltpu.SemaphoreType.DMA,
        pltpu.SemaphoreType.DMA,
    ],
    compiler_params=pltpu.CompilerParams(use_tc_tiling_on_sc=False),
)
def _sc_agg(row_hbm, col_hbm, z_hbm, out_hbm,
            idxrb, idxcb, rows0, rows1, acc_sh, sem0, sem1):
    c = lax.axis_index("c")
    s = lax.axis_index("s")

    def zrow(i, _):
        for k in range(D // 16):
            rows0[i, pl.ds(k * 16, 16)] = jnp.zeros((16,), jnp.float32)
        return 0

    lax.fori_loop(0, CH, zrow, 0)
    for q in range(STRIPE // CH):
        pltpu.sync_copy(rows0, acc_sh.at[pl.ds(s * STRIPE + q * CH, CH)])
    pltpu.sync_copy(rows0.at[pl.ds(0, STRIPE % CH)],
                    acc_sh.at[pl.ds(s * STRIPE + (STRIPE // CH) * CH,
                                    STRIPE % CH)])
    plsc.subcore_barrier()

    lo = c * HALF

    def jbody(j, _):
        b = (s * CPT + j * IBLK) * CH
        pltpu.sync_copy(row_hbm.at[pl.ds(b, IBLK * CH)], idxrb)
        pltpu.sync_copy(col_hbm.at[pl.ds(b, IBLK * CH)], idxcb)

        def rbody(k, _):
            cv = idxcb[pl.ds(k * 16, 16)]
            lc = cv - lo
            ok = (lc >= 0) & (lc < HALF)
            idxcb[pl.ds(k * 16, 16)] = jnp.where(ok, lc, DUMMY)
            return 0

        lax.fori_loop(0, IBLK * CH // 16, rbody, 0)
        bufs = (rows0, rows1)
        sems = (sem0, sem1)

        def gather(k, buf, sem):
            return pltpu.async_copy(z_hbm.at[idxrb.at[pl.ds(k * CH, CH)]],
                                    buf, sem)

        def scatter(k, buf):
            pltpu.sync_copy(buf, acc_sh.at[idxcb.at[pl.ds(k * CH, CH)]],
                            add=True)

        h = [gather(0, rows0, sem0), gather(1, rows1, sem1)]
        for k in range(2, IBLK):
            q = k % 2
            h[q].wait()
            scatter(k - 2, bufs[q])
            h[q] = gather(k, bufs[q], sems[q])
        h[0].wait()
        scatter(IBLK - 2, rows0)
        h[1].wait()
        scatter(IBLK - 1, rows1)
        return 0

    lax.fori_loop(0, CPT // IBLK, jbody, 0)

    plsc.subcore_barrier()
    for q in range(STRIPE // CH):
        st = s * STRIPE + q * CH
        pltpu.sync_copy(acc_sh.at[pl.ds(st, CH)], rows0)
        pltpu.sync_copy(rows0, out_hbm.at[c, pl.ds(st, CH)])
    st = s * STRIPE + (STRIPE // CH) * CH
    pltpu.sync_copy(acc_sh.at[pl.ds(st, STRIPE % CH)],
                    rows0.at[pl.ds(0, STRIPE % CH)])
    pltpu.sync_copy(rows0.at[pl.ds(0, STRIPE % CH)],
                    out_hbm.at[c, pl.ds(st, STRIPE % CH)])


BLK = 1000
GRID = N // BLK


def _tc_pre_body(e_ref, d0_ref, d1_ref, wt_ref, b_ref, dinv_ref, z_ref):
    dinv = lax.rsqrt(d0_ref[...] + d1_ref[...] + 1.0)
    y = jnp.dot(e_ref[...], wt_ref[...], preferred_element_type=jnp.float32)
    dinv_ref[...] = dinv
    z_ref[...] = dinv * (y + b_ref[...])


def _tc_mid_body(acc_ref, z_ref, dinv_ref, wt_ref, b_ref, e_ref, zn_ref):
    o = dinv_ref[...] * (acc_ref[...] + z_ref[...])
    o = jnp.where(o >= 0, o, 0.01 * o)
    nrm = jnp.sqrt(jnp.sum(o * o, axis=1, keepdims=True))
    o = o / jnp.maximum(nrm, 1e-12)
    e_ref[...] = o
    zn_ref[...] = dinv_ref[...] * (
        jnp.dot(o, wt_ref[...], preferred_element_type=jnp.float32) + b_ref[...])


def _tc_post_body(acc_ref, z_ref, dinv_ref, e_ref):
    o = dinv_ref[...] * (acc_ref[...] + z_ref[...])
    o = jnp.where(o >= 0, o, 0.01 * o)
    nrm = jnp.sqrt(jnp.sum(o * o, axis=1, keepdims=True))
    e_ref[...] = o / jnp.maximum(nrm, 1e-12)


_row_spec = pl.BlockSpec((BLK, D), lambda i: (i, 0))
_col1_spec = pl.BlockSpec((BLK, 1), lambda i: (i, 0))
_w_spec = pl.BlockSpec((D, D), lambda i: (0, 0))
_b_spec = pl.BlockSpec((1, D), lambda i: (0, 0))

_tc_pre = pl.pallas_call(
    _tc_pre_body,
    grid=(GRID,),
    in_specs=[_row_spec, _col1_spec, _col1_spec, _w_spec, _b_spec],
    out_specs=[_col1_spec, _row_spec],
    out_shape=[
        jax.ShapeDtypeStruct((N, 1), jnp.float32),
        jax.ShapeDtypeStruct((N, D), jnp.float32),
    ],
)

_tc_mid = pl.pallas_call(
    _tc_mid_body,
    grid=(GRID,),
    in_specs=[_row_spec, _row_spec, _col1_spec, _w_spec, _b_spec],
    out_specs=[_row_spec, _row_spec],
    out_shape=[
        jax.ShapeDtypeStruct((N, D), jnp.float32),
        jax.ShapeDtypeStruct((N, D), jnp.float32),
    ],
)

_tc_post = pl.pallas_call(
    _tc_post_body,
    grid=(GRID,),
    in_specs=[_row_spec, _row_spec, _col1_spec],
    out_specs=_row_spec,
    out_shape=jax.ShapeDtypeStruct((N, D), jnp.float32),
)


def kernel(edge_index, E, W1, b1, W2, b2, W3, b3):
    ei = edge_index.astype(jnp.int32)
    pad = EPAD - EDGES
    row = jnp.concatenate([ei[0], jnp.zeros((pad,), jnp.int32)])
    col = jnp.concatenate([ei[1], jnp.full((pad,), N, jnp.int32)])

    deg2 = _sc_deg(col)
    d0 = deg2[:N, None]
    d1 = deg2[DEG_R:DEG_R + N, None]

    dinv, z1 = _tc_pre(E, d0, d1, W1.T, b1[None, :])

    def agg(z):
        a = _sc_agg(row, col, z)
        return jnp.concatenate([a[0, :HALF], a[1, :HALF]], axis=0)

    E1, z2 = _tc_mid(agg(z1), z1, dinv, W2.T, b2[None, :])
    E2, z3 = _tc_mid(agg(z2), z2, dinv, W3.T, b3[None, :])
    E3 = _tc_post(agg(z3), z3, dinv)
    return jnp.concatenate([E, E1, E2, E3], axis=1)
